# Initial kernel scaffold; baseline (speedup 1.0000x reference)
import jax
import jax.numpy as jnp
from jax.experimental import pallas as pl


def kernel(x, params, edge0, edge1, edge2, edge3, edge4, pool1, pool2, pool3, pool4):
    raise NotImplementedError("write your pallas kernel here")


# speedup vs baseline: 2.1313x; 2.1313x over previous
import functools

import jax
import jax.numpy as jnp
from jax import lax
from jax.experimental import pallas as pl
from jax.experimental.pallas import tpu as pltpu
from jax.experimental.pallas import tpu_sc as plsc

NC = 2
NSUB = 16
NW = NC * NSUB
ECH = 128
TILE = 256
SPMEM_BUDGET = 11 * 512 * 1024

_CS = [32, 32, 64, 128, 256, 256, 128, 96, 96]
_NS = [50000, 25000, 12500, 6250, 3125]


def _pad256(n):
    return -(-n // 256) * 256


def _pad4096(n):
    return -(-n // 4096) * 4096


def _pick_cc(C, n_pad):
    for k in range(1, C // 16 + 1):
        if C % k:
            continue
        cc = C // k
        if cc % 16:
            continue
        if n_pad * cc * 4 <= SPMEM_BUDGET:
            return cc
    return 16


def _sc_scatter(x_slice, src, dst, zeros, n_out_pad, cc, e_pad):
    e_w = e_pad // NW
    n_iter = e_w // ECH
    r_s = n_out_pad // NSUB
    mesh = plsc.VectorSubcoreMesh(core_axis_name="c", subcore_axis_name="s",
                                  num_cores=NC, num_subcores=NSUB)

    def body(x_hbm, src_hbm, dst_hbm, z_hbm, out_hbm, sidx, didx, rows, acc, sem):
        c = lax.axis_index("c")
        s = lax.axis_index("s")
        wid = s * NC + c
        sl = pl.ds(s * r_s, r_s)
        pltpu.sync_copy(z_hbm.at[sl], acc.at[sl])
        plsc.subcore_barrier()
        wbase = wid * e_w

        def step(i, carry):
            base = wbase + i * ECH
            pltpu.sync_copy(src_hbm.at[pl.ds(base, ECH)], sidx)
            pltpu.sync_copy(dst_hbm.at[pl.ds(base, ECH)], didx)
            pltpu.async_copy(x_hbm.at[sidx], rows, sem).wait()
            pltpu.sync_copy(rows, acc.at[didx], add=True)
            return carry

        lax.fori_loop(0, n_iter, step, 0)
        plsc.subcore_barrier()

        @pl.when(c == 0)
        def _():
            pltpu.sync_copy(acc.at[sl], out_hbm.at[0].at[sl])

        @pl.when(c == 1)
        def _():
            pltpu.sync_copy(acc.at[sl], out_hbm.at[1].at[sl])

    f = pl.kernel(
        body,
        out_type=jax.ShapeDtypeStruct((2, n_out_pad, cc), jnp.float32),
        mesh=mesh,
        scratch_types=[
            pltpu.VMEM((ECH,), jnp.int32),
            pltpu.VMEM((ECH,), jnp.int32),
            pltpu.VMEM((ECH, cc), jnp.float32),
            pltpu.VMEM_SHARED((n_out_pad, cc), jnp.float32),
            pltpu.SemaphoreType.DMA,
        ],
        compiler_params=pltpu.CompilerParams(use_tc_tiling_on_sc=False),
    )
    return f(x_slice, src, dst, zeros)


def _sc_gather(tab, idx, n_idx_pad):
    C = tab.shape[1]
    r_w = n_idx_pad // NW
    n_iter = r_w // ECH
    mesh = plsc.VectorSubcoreMesh(core_axis_name="c", subcore_axis_name="s",
                                  num_cores=NC, num_subcores=NSUB)

    def body(tab_hbm, idx_hbm, out_hbm, iv, rows, sem):
        c = lax.axis_index("c")
        s = lax.axis_index("s")
        wid = s * NC + c

        def step(i, carry):
            base = wid * r_w + i * ECH
            pltpu.sync_copy(idx_hbm.at[pl.ds(base, ECH)], iv)
            pltpu.async_copy(tab_hbm.at[iv], rows, sem).wait()
            pltpu.sync_copy(rows, out_hbm.at[pl.ds(base, ECH)])
            return carry

        lax.fori_loop(0, n_iter, step, 0)

    f = pl.kernel(
        body,
        out_type=jax.ShapeDtypeStruct((n_idx_pad, C), jnp.float32),
        mesh=mesh,
        scratch_types=[
            pltpu.VMEM((ECH,), jnp.int32),
            pltpu.VMEM((ECH, C), jnp.float32),
            pltpu.SemaphoreType.DMA,
        ],
        compiler_params=pltpu.CompilerParams(use_tc_tiling_on_sc=False),
    )
    return f(tab, idx)


def _mp_parts(x, src, dst, n_out_pad, C, e_pad):
    cc = _pick_cc(C, n_out_pad)
    z = jnp.zeros((n_out_pad, cc), jnp.float32)
    parts = []
    for k0 in range(0, C, cc):
        xs = x[:, k0:k0 + cc]
        parts.append(_sc_scatter(xs, src, dst, z, n_out_pad, cc, e_pad))
    return parts, cc


def _mm_stats(parts, cc, x_opt, W, b, n_true, n_pad):
    K, C = W.shape
    nt = n_pad // TILE
    nparts = len(parts)
    has_x = x_opt is not None

    def body(*refs):
        i = pl.program_id(0)
        p_refs = refs[:nparts]
        off = nparts
        x_ref = refs[off] if has_x else None
        off += 1 if has_x else 0
        w_ref, b_ref = refs[off], refs[off + 1]
        y_ref, st_ref = refs[off + 2], refs[off + 3]
        acc = refs[off + 4]
        cols = []
        for pr in p_refs:
            pv = pr[...]
            cols.append(pv[0] + pv[1])
        agg = None
        if cols:
            agg = cols[0] if len(cols) == 1 else jnp.concatenate(cols, axis=1)
        if has_x:
            agg = x_ref[...] if agg is None else agg + x_ref[...]
        rows = i * TILE + lax.broadcasted_iota(jnp.int32, (TILE, 1), 0)
        valid = rows < n_true
        agg = jnp.where(valid, agg, 0.0)
        y = jnp.dot(agg, w_ref[...], preferred_element_type=jnp.float32) + b_ref[...]
        y = jnp.where(valid, y, 0.0)

        @pl.when(i == 0)
        def _():
            acc[...] = jnp.zeros_like(acc)

        acc[0, :] += jnp.sum(y, axis=0)
        acc[1, :] += jnp.sum(y * y, axis=0)
        y_ref[...] = y

        @pl.when(i == nt - 1)
        def _():
            st_ref[...] = acc[...]

    in_specs = (
        [pl.BlockSpec((2, TILE, cc), lambda i: (0, i, 0)) for _ in parts]
        + ([pl.BlockSpec((TILE, K), lambda i: (i, 0))] if has_x else [])
        + [pl.BlockSpec((K, C), lambda i: (0, 0)),
           pl.BlockSpec((1, C), lambda i: (0, 0))]
    )
    out_specs = [
        pl.BlockSpec((TILE, C), lambda i: (i, 0)),
        pl.BlockSpec((8, C), lambda i: (0, 0)),
    ]
    args = list(parts) + ([x_opt] if has_x else []) + [W, b.reshape(1, C)]
    y, st = pl.pallas_call(
        body,
        grid=(nt,),
        in_specs=in_specs,
        out_specs=out_specs,
        out_shape=[
            jax.ShapeDtypeStruct((n_pad, C), jnp.float32),
            jax.ShapeDtypeStruct((8, C), jnp.float32),
        ],
        scratch_shapes=[pltpu.VMEM((8, C), jnp.float32)],
        compiler_params=pltpu.CompilerParams(
            dimension_semantics=("arbitrary",)),
    )(*args)
    return y, st


def _bn(y, st, g, be, n_true, relu=True, second=None, xres=None):
    n_pad, C = y.shape
    nt = n_pad // TILE
    has2 = second is not None
    hasx = xres is not None

    def body(*refs):
        i = pl.program_id(0)
        y_ref, st_ref, g_ref, be_ref = refs[0:4]
        off = 4

        def norm(yv, stv, gv, bev):
            mu = stv[0] / n_true
            var = stv[1] / n_true - mu * mu
            return (yv - mu) * (gv * lax.rsqrt(var + 1e-5)) + bev

        out = norm(y_ref[...], st_ref[...], g_ref[...], be_ref[...])
        if has2:
            y2_ref, st2_ref, g2_ref, be2_ref = refs[off:off + 4]
            off += 4
            out = out + norm(y2_ref[...], st2_ref[...], g2_ref[...], be2_ref[...])
        if hasx:
            out = out + refs[off][...]
            off += 1
        if relu:
            out = jnp.maximum(out, 0.0)
        rows = i * TILE + lax.broadcasted_iota(jnp.int32, (TILE, 1), 0)
        out = jnp.where(rows < n_true, out, 0.0)
        refs[off][...] = out

    y_spec = pl.BlockSpec((TILE, C), lambda i: (i, 0))
    st_spec = pl.BlockSpec((8, C), lambda i: (0, 0))
    v_spec = pl.BlockSpec((1, C), lambda i: (0, 0))
    in_specs = [y_spec, st_spec, v_spec, v_spec]
    args = [y, st, g.reshape(1, C), be.reshape(1, C)]
    if has2:
        y2, st2, g2, be2 = second
        in_specs += [y_spec, st_spec, v_spec, v_spec]
        args += [y2, st2, g2.reshape(1, C), be2.reshape(1, C)]
    if hasx:
        in_specs += [y_spec]
        args += [xres]
    return pl.pallas_call(
        body,
        grid=(nt,),
        in_specs=in_specs,
        out_specs=y_spec,
        out_shape=jax.ShapeDtypeStruct((n_pad, C), jnp.float32),
        compiler_params=pltpu.CompilerParams(
            dimension_semantics=("arbitrary",)),
    )(*args)


def _cbr_op(x, src, dst, e_pad, p, n_pad, n_true):
    C_in = x.shape[1]
    parts, cc = _mp_parts(x, src, dst, n_pad, C_in, e_pad)
    y, st = _mm_stats(parts, cc, x, p["W"], p["b"], n_true, n_pad)
    return _bn(y, st, p["g"], p["be"], n_true, relu=True)


def _res_op(x, src, dst, e_pad, p, n_pad, n_true):
    parts, cc = _mp_parts(x, src, dst, n_pad, x.shape[1], e_pad)
    y1, st1 = _mm_stats(parts, cc, x, p["c1"]["W"], p["c1"]["b"], n_true, n_pad)
    h = _bn(y1, st1, p["c1"]["g"], p["c1"]["be"], n_true, relu=True)
    parts2, cc2 = _mp_parts(h, src, dst, n_pad, h.shape[1], e_pad)
    y2, st2 = _mm_stats(parts2, cc2, h, p["c2"]["W"], p["c2"]["b"], n_true, n_pad)
    if "sc" in p:
        ysc, stsc = _mm_stats([], 0, x, p["sc"]["W"], p["sc"]["b"], n_true, n_pad)
        return _bn(y2, st2, p["c2"]["g"], p["c2"]["be"], n_true, relu=True,
                   second=(ysc, stsc, p["sc"]["g"], p["sc"]["be"]))
    return _bn(y2, st2, p["c2"]["g"], p["c2"]["be"], n_true, relu=True, xres=x)


def _down_op(x, pool, p, n_f_true, n_c_pad, n_c_true):
    C = x.shape[1]
    e_pad = _pad4096(n_f_true)
    ar = jnp.arange(n_f_true, dtype=jnp.int32)
    src = jnp.concatenate([ar, jnp.zeros((e_pad - n_f_true,), jnp.int32)])
    dst = jnp.concatenate(
        [pool, jnp.full((e_pad - n_f_true,), n_c_true, jnp.int32)])
    parts, cc = _mp_parts(x, src, dst, n_c_pad, C, e_pad)
    y, st = _mm_stats(parts, cc, None, p["W"], p["b"], n_c_true, n_c_pad)
    return _bn(y, st, p["g"], p["be"], n_c_true, relu=True)


def _up_op(x, pool, p, n_c_true, n_f_true, n_f_pad):
    n_c_pad = x.shape[0]
    y, st = _mm_stats([], 0, x, p["W"], p["b"], n_c_true, n_c_pad)
    h = _bn(y, st, p["g"], p["be"], n_c_true, relu=True)
    e_pad = _pad4096(n_f_true)
    idx = jnp.concatenate([pool, jnp.zeros((e_pad - n_f_true,), jnp.int32)])
    out = _sc_gather(h, idx, e_pad)
    return out[:n_f_pad]


def _pad_edges(e, n_dst):
    E = e.shape[1]
    ep = _pad4096(E)
    src = jnp.concatenate([e[0], jnp.zeros((ep - E,), jnp.int32)])
    dst = jnp.concatenate([e[1], jnp.full((ep - E,), n_dst, jnp.int32)])
    return src, dst, ep


def kernel(x, params, edge0, edge1, edge2, edge3, edge4, pool1, pool2, pool3, pool4):
    P = params
    npad = [_pad256(n) for n in _NS]
    edges = [_pad_edges(e, _NS[l])
             for l, e in enumerate([edge0, edge1, edge2, edge3, edge4])]
    pools = [pool1, pool2, pool3, pool4]

    x0 = jnp.pad(x, ((0, npad[0] - _NS[0]), (0, 12)))
    stem1_W = jnp.pad(P["stem1"]["W"], ((0, 12), (0, 0)))
    stem1 = dict(P["stem1"], W=stem1_W)

    s0, d0, ep0 = edges[0]
    stem = _cbr_op(x0, s0, d0, ep0, stem1, npad[0], _NS[0])
    stem = _cbr_op(stem, s0, d0, ep0, P["stem2"], npad[0], _NS[0])

    skips = [stem]
    cur = stem
    for l in range(1, 5):
        cur = _down_op(cur, pools[l - 1], P["down%d" % l],
                       _NS[l - 1], npad[l], _NS[l])
        s, dd, ep = edges[l]
        cur = _res_op(cur, s, dd, ep, P["s%dr1" % l], npad[l], _NS[l])
        cur = _res_op(cur, s, dd, ep, P["s%dr2" % l], npad[l], _NS[l])
        if l < 4:
            skips.append(cur)

    for u in range(1, 5):
        lvl = 4 - u
        up = _up_op(cur, pools[lvl], P["up%dde" % u],
                    _NS[lvl + 1], _NS[lvl], npad[lvl])
        y = jnp.concatenate([up, skips[lvl]], axis=1)
        s, dd, ep = edges[lvl]
        cur = _res_op(y, s, dd, ep, P["u%dr1" % u], npad[lvl], _NS[lvl])
        cur = _res_op(cur, s, dd, ep, P["u%dr2" % u], npad[lvl], _NS[lvl])

    return cur[:_NS[0]]
